# Initial kernel scaffold; baseline (speedup 1.0000x reference)
import jax
import jax.numpy as jnp
from jax.experimental import pallas as pl


def kernel(x_s, x_t, edge_index_s0, edge_index_s1, edge_index_t0, edge_index_t1, W1l, W1r, b1, bn1_g, bn1_b, W2l, W2r, b2, bn2_g, bn2_b, cls_W1, cls_b1, cls_W2, cls_b2, dom_W1, dom_b1, dom_bn_g, dom_bn_b, dom_W2, dom_b2):
    raise NotImplementedError("write your pallas kernel here")


# speedup vs baseline: 2.8196x; 2.8196x over previous
import jax
import jax.numpy as jnp
from jax import lax
from jax.experimental import pallas as pl
from jax.experimental.pallas import tpu as pltpu
from jax.experimental.pallas import tpu_sc as plsc

N = 10000
E = 320000
D = 128
EPS = 1e-5

NC = 2
NS = 16
NW = NC * NS
CHUNK = 128
NCHB = 8
NBLK = 10
NCH = NCHB * NBLK
EPAD = NW * NCH * CHUNK
ROWS_PER_TILE = 640
NPAD = NS * ROWS_PER_TILE
OROWS = 16
ONC = ROWS_PER_TILE // OROWS
CROWS = NPAD // 128
NSET = 4


def _seg_kernel_body(x_hbm, srcs, dsts, agg_out, acc_s, sem):
    cid = lax.axis_index("c")
    sid = lax.axis_index("s")
    wid = sid * NC + cid
    base = sid * ROWS_PER_TILE

    def _inner(src_v, dst_v, buf, obuf):
        def _zrow(i, _):
            def _z16(j, _):
                obuf[i, pl.ds(j * 16, 16)] = jnp.zeros((16,), jnp.float32)
                return 0
            lax.fori_loop(0, D // 16, _z16, 0)
            return 0
        lax.fori_loop(0, OROWS, _zrow, 0)

        def _zs(k, _):
            pltpu.sync_copy(obuf, acc_s.at[pl.ds(base + k * OROWS, OROWS)])
            return 0
        lax.fori_loop(0, ONC, _zs, 0)

        plsc.subcore_barrier()

        def _blk(bi, _):
            pltpu.sync_copy(srcs.at[wid, pl.ds(bi * NCHB, NCHB)], src_v)
            pltpu.sync_copy(dsts.at[wid, pl.ds(bi * NCHB, NCHB)], dst_v)

            def _chunk(j, _):
                pltpu.async_copy(x_hbm.at[src_v.at[j]], buf, sem).wait()
                pltpu.sync_copy(buf, acc_s.at[dst_v.at[j]], add=True)
                return 0
            lax.fori_loop(0, NCHB, _chunk, 0)
            return 0
        lax.fori_loop(0, NBLK, _blk, 0)

        plsc.subcore_barrier()

        def _out(k, _):
            off = base + k * OROWS
            pltpu.sync_copy(acc_s.at[pl.ds(off, OROWS)], obuf)
            pltpu.sync_copy(obuf, agg_out.at[cid, pl.ds(off, OROWS)])
            return 0
        lax.fori_loop(0, ONC, _out, 0)

    pl.run_scoped(
        _inner,
        src_v=pltpu.VMEM((NCHB, CHUNK), jnp.int32),
        dst_v=pltpu.VMEM((NCHB, CHUNK), jnp.int32),
        buf=pltpu.VMEM((CHUNK, D), jnp.float32),
        obuf=pltpu.VMEM((OROWS, D), jnp.float32),
    )


_seg_sum = pl.kernel(
    _seg_kernel_body,
    out_type=jax.ShapeDtypeStruct((NC, NPAD, D), jnp.float32),
    mesh=plsc.VectorSubcoreMesh(core_axis_name="c", subcore_axis_name="s"),
    scratch_types=[
        pltpu.VMEM_SHARED((NPAD, D), jnp.float32),
        pltpu.SemaphoreType.DMA,
    ],
)


def _deg_kernel_body(dsts4, cnt_out, sem):
    cid = lax.axis_index("c")
    sid = lax.axis_index("s")
    wid = sid * NC + cid

    def _inner(dst_v, cnt_v):
        ones16 = jnp.ones((16,), jnp.float32)

        def _set(e, _):
            def _zrow(i, _):
                def _z16(j, _):
                    cnt_v[i, pl.ds(j * 16, 16)] = jnp.zeros((16,), jnp.float32)
                    return 0
                lax.fori_loop(0, D // 16, _z16, 0)
                return 0
            lax.fori_loop(0, CROWS, _zrow, 0)

            def _blk(bi, _):
                pltpu.sync_copy(dsts4.at[e, wid, pl.ds(bi * NCHB, NCHB)], dst_v)

                def _chunk(j, _):
                    def _grp(k, _):
                        idx = dst_v[j, pl.ds(k * 16, 16)]
                        row = lax.shift_right_logical(idx, 7)
                        col = lax.bitwise_and(idx, 127)
                        plsc.addupdate_scatter(cnt_v, [row, col], ones16)
                        return 0
                    lax.fori_loop(0, CHUNK // 16, _grp, 0)
                    return 0
                lax.fori_loop(0, NCHB, _chunk, 0)
                return 0
            lax.fori_loop(0, NBLK, _blk, 0)

            pltpu.sync_copy(cnt_v, cnt_out.at[e, wid])
            return 0
        lax.fori_loop(0, NSET, _set, 0)

    pl.run_scoped(
        _inner,
        dst_v=pltpu.VMEM((NCHB, CHUNK), jnp.int32),
        cnt_v=pltpu.VMEM((CROWS, 128), jnp.float32),
    )


_deg = pl.kernel(
    _deg_kernel_body,
    out_type=jax.ShapeDtypeStruct((NSET, NW, CROWS, 128), jnp.float32),
    mesh=plsc.VectorSubcoreMesh(core_axis_name="c", subcore_axis_name="s"),
    scratch_types=[pltpu.SemaphoreType.DMA],
    compiler_params=pltpu.CompilerParams(needs_layout_passes=False),
)


def _cntsum_body(cntp, out):
    out[...] = jnp.sum(cntp[...], axis=1)


_cntsum = pl.pallas_call(
    _cntsum_body,
    out_shape=jax.ShapeDtypeStruct((NSET, CROWS, 128), jnp.float32),
)


def _sage_tc_body(aggp, denom, x, Wl, Wr, b, g, bb, out):
    agg = aggp[0, :N] + aggp[1, :N]
    z = (jnp.dot(agg / denom[...], Wl[...], preferred_element_type=jnp.float32)
         + jnp.dot(x[...], Wr[...], preferred_element_type=jnp.float32)
         + b[...])
    mu = jnp.mean(z, axis=0, keepdims=True)
    var = jnp.mean((z - mu) ** 2, axis=0, keepdims=True)
    h = (z - mu) / jnp.sqrt(var + EPS) * g[...] + bb[...]
    out[...] = jnp.maximum(h, 0.0)


_sage_tc = pl.pallas_call(
    _sage_tc_body,
    out_shape=jax.ShapeDtypeStruct((N, D), jnp.float32),
)


def _heads_body(f, cW1, cb1, cW2, cb2, dW1, db1, dg, db, dW2, db2,
                pred, dom, fmean):
    fv = f[...]
    h1 = jnp.maximum(jnp.dot(fv, cW1[...], preferred_element_type=jnp.float32)
                     + cb1[...], 0.0)
    pred[...] = jnp.dot(h1, cW2[...], preferred_element_type=jnp.float32) + cb2[...]

    zd = jnp.dot(fv, dW1[...], preferred_element_type=jnp.float32) + db1[...]
    mu = jnp.mean(zd, axis=0, keepdims=True)
    var = jnp.mean((zd - mu) ** 2, axis=0, keepdims=True)
    zn = (zd - mu) / jnp.sqrt(var + EPS) * dg[...] + db[...]
    dom[...] = (jnp.dot(jnp.maximum(zn, 0.0), dW2[...],
                        preferred_element_type=jnp.float32) + db2[...])
    fmean[...] = jnp.mean(fv, axis=0, keepdims=True)


_heads = pl.pallas_call(
    _heads_body,
    out_shape=(
        jax.ShapeDtypeStruct((N, 10), jnp.float32),
        jax.ShapeDtypeStruct((N, 2), jnp.float32),
        jax.ShapeDtypeStruct((1, D), jnp.float32),
    ),
)


def _mmd_body(ms, mt, out):
    d = ms[...] - mt[...]
    out[...] = jnp.sum(d * d, keepdims=True).reshape(1, 1)


_mmd = pl.pallas_call(
    _mmd_body,
    out_shape=jax.ShapeDtypeStruct((1, 1), jnp.float32),
)


def _prep_edges(ei):
    src = ei[0]
    dst = ei[1]
    pad = EPAD - E
    src = jnp.concatenate([src, jnp.zeros((pad,), jnp.int32)])
    dst = jnp.concatenate([dst, jnp.full((pad,), N, jnp.int32)])
    return src.reshape(NW, NCH, CHUNK), dst.reshape(NW, NCH, CHUNK)


def _extract(x, s0, d0, s1, d1, den0, den1,
             W1l, W1r, b1, g1, bb1, W2l, W2r, b2, g2, bb2):
    agg0 = _seg_sum(x, s0, d0)
    h = _sage_tc(agg0, den0, x, W1l, W1r, b1, g1, bb1)
    agg1 = _seg_sum(h, s1, d1)
    return _sage_tc(agg1, den1, h, W2l, W2r, b2, g2, bb2)


def kernel(x_s, x_t, edge_index_s0, edge_index_s1, edge_index_t0, edge_index_t1,
           W1l, W1r, b1, bn1_g, bn1_b, W2l, W2r, b2, bn2_g, bn2_b,
           cls_W1, cls_b1, cls_W2, cls_b2,
           dom_W1, dom_b1, dom_bn_g, dom_bn_b, dom_W2, dom_b2):
    row = lambda v: v.reshape(1, -1)
    b1r, g1r, bb1r = row(b1), row(bn1_g), row(bn1_b)
    b2r, g2r, bb2r = row(b2), row(bn2_g), row(bn2_b)

    edges = [_prep_edges(e) for e in (edge_index_s0, edge_index_s1,
                                      edge_index_t0, edge_index_t1)]
    dsts4 = jnp.stack([d for _, d in edges])
    cnt_p = _deg(dsts4)
    cnt = _cntsum(cnt_p)
    dens = jnp.maximum(cnt.reshape(NSET, NPAD, 1)[:, :N], 1.0)

    fs = _extract(x_s, edges[0][0], edges[0][1], edges[1][0], edges[1][1],
                  dens[0], dens[1],
                  W1l, W1r, b1r, g1r, bb1r, W2l, W2r, b2r, g2r, bb2r)
    ft = _extract(x_t, edges[2][0], edges[2][1], edges[3][0], edges[3][1],
                  dens[2], dens[3],
                  W1l, W1r, b1r, g1r, bb1r, W2l, W2r, b2r, g2r, bb2r)

    s_pred, s_dom, ms = _heads(fs, cls_W1, row(cls_b1), cls_W2, row(cls_b2),
                               dom_W1, row(dom_b1), row(dom_bn_g),
                               row(dom_bn_b), dom_W2, row(dom_b2))
    t_pred, t_dom, mt = _heads(ft, cls_W1, row(cls_b1), cls_W2, row(cls_b2),
                               dom_W1, row(dom_b1), row(dom_bn_g),
                               row(dom_bn_b), dom_W2, row(dom_b2))
    loss_mmd = _mmd(ms, mt)[0, 0]
    return (s_pred, t_pred, s_dom, t_dom, loss_mmd)
